# Initial kernel scaffold; baseline (speedup 1.0000x reference)
import jax
import jax.numpy as jnp
from jax.experimental import pallas as pl


def kernel(x, edge_index, prev_hidden, params):
    raise NotImplementedError("write your pallas kernel here")


# speedup vs baseline: 3.5783x; 3.5783x over previous
import functools

import jax
import jax.numpy as jnp
from jax import lax
from jax.experimental import pallas as pl
from jax.experimental.pallas import tpu as pltpu
from jax.experimental.pallas import tpu_sc as plsc

_LANES = 128


def _sc_gather(table, idx, chunk):
    _, d = table.shape
    b = idx.shape[0]
    info = plsc.get_sparse_core_info()
    nc = info.num_cores
    nw = nc * info.num_subcores
    b_per_w = b // nw
    assert b_per_w * nw == b and b_per_w % chunk == 0 and chunk % 8 == 0
    n_chunks = b_per_w // chunk
    mesh = plsc.VectorSubcoreMesh(core_axis_name="c", subcore_axis_name="s")

    @functools.partial(
        pl.kernel,
        mesh=mesh,
        out_type=jax.ShapeDtypeStruct((b, d), table.dtype),
        scratch_types=[
            pltpu.VMEM((chunk,), jnp.int32),
            pltpu.VMEM((chunk, d), table.dtype),
            pltpu.SemaphoreType.DMA,
        ],
    )
    def gather_kernel(table_hbm, idx_hbm, out_hbm, idx_v, rows_v, sem):
        wid = lax.axis_index("s") * nc + lax.axis_index("c")
        base = wid * b_per_w

        def body(ci, carry):
            off = base + ci * chunk
            pltpu.sync_copy(idx_hbm.at[pl.ds(off, chunk)], idx_v)
            pltpu.async_copy(table_hbm.at[idx_v], rows_v, sem).wait()
            pltpu.sync_copy(rows_v, out_hbm.at[pl.ds(off, chunk)])
            return carry

        lax.fori_loop(0, n_chunks, body, 0)

    return gather_kernel(table, idx)


def _sage_layer(feats, m_t, p, gru, relu, block):
    n = feats.shape[0]
    deg = m_t.shape[0]
    d_in, d_out = p["W_self"].shape
    f32 = jnp.float32

    def body(*refs):
        if gru is not None:
            (f_ref, m_ref, wx_ref, wh_ref, bl_ref, ws_ref, bs_ref, wn_ref,
             hprev_ref, wih_ref, whh_ref, bih_ref, bhh_ref, out_ref) = refs
        else:
            (f_ref, m_ref, wx_ref, wh_ref, bl_ref, ws_ref, bs_ref, wn_ref,
             out_ref) = refs
        f = f_ref[...][:, :d_in]
        wx = wx_ref[...]
        wh = wh_ref[...]
        bl = bl_ref[...]
        h = jnp.zeros((block, d_in), f32)
        c = jnp.zeros((block, d_in), f32)
        for t in range(deg):
            z = (jnp.dot(m_ref[t][:, :d_in], wx, preferred_element_type=f32)
                 + jnp.dot(h, wh, preferred_element_type=f32) + bl)
            zi = z[:, :d_in]
            zf = z[:, d_in:2 * d_in]
            zg = z[:, 2 * d_in:3 * d_in]
            zo = z[:, 3 * d_in:]
            c = jax.nn.sigmoid(zf) * c + jax.nn.sigmoid(zi) * jnp.tanh(zg)
            h = jax.nn.sigmoid(zo) * jnp.tanh(c)
        out = (jnp.dot(f, ws_ref[...], preferred_element_type=f32)
               + bs_ref[...]
               + jnp.dot(h, wn_ref[...], preferred_element_type=f32))
        if relu:
            out = jnp.maximum(out, 0.0)
        if gru is not None:
            hp = hprev_ref[...]
            dh = hp.shape[1]
            og = out[:, :dh]
            gi = (jnp.dot(og, wih_ref[...], preferred_element_type=f32)
                  + bih_ref[...])
            gh = (jnp.dot(hp, whh_ref[...], preferred_element_type=f32)
                  + bhh_ref[...])
            r = jax.nn.sigmoid(gi[:, :dh] + gh[:, :dh])
            zz = jax.nn.sigmoid(gi[:, dh:2 * dh] + gh[:, dh:2 * dh])
            nw = jnp.tanh(gi[:, 2 * dh:] + r * gh[:, 2 * dh:])
            og = (1.0 - zz) * nw + zz * hp
            out = jnp.concatenate(
                [og, jnp.zeros((block, _LANES - dh), f32)], axis=1)
        out_ref[...] = out

    def full(shape):
        return pl.BlockSpec(shape, lambda i: (0,) * len(shape))

    in_specs = [
        pl.BlockSpec((block, _LANES), lambda i: (i, 0)),
        pl.BlockSpec((deg, block, _LANES), lambda i: (0, i, 0)),
        full((d_in, 4 * d_in)),
        full((d_in, 4 * d_in)),
        full((1, 4 * d_in)),
        full((d_in, _LANES)),
        full((1, _LANES)),
        full((d_in, _LANES)),
    ]
    pad = ((0, 0), (0, _LANES - d_out))
    args = [feats, m_t, p["Wx"], p["Wh"], p["b_lstm"].reshape(1, -1),
            jnp.pad(p["W_self"], pad),
            jnp.pad(p["b_self"].reshape(1, -1), ((0, 0), (0, _LANES - d_out))),
            jnp.pad(p["W_neigh"], pad)]
    if gru is not None:
        hprev, gp = gru
        dh = hprev.shape[1]
        in_specs += [
            pl.BlockSpec((block, dh), lambda i: (i, 0)),
            full((dh, 3 * dh)),
            full((dh, 3 * dh)),
            full((1, 3 * dh)),
            full((1, 3 * dh)),
        ]
        args += [hprev, gp["W_ih"], gp["W_hh"], gp["b_ih"].reshape(1, -1),
                 gp["b_hh"].reshape(1, -1)]

    return pl.pallas_call(
        body,
        grid=(n // block,),
        in_specs=in_specs,
        out_specs=pl.BlockSpec((block, _LANES), lambda i: (i, 0)),
        out_shape=jax.ShapeDtypeStruct((n, _LANES), f32),
        compiler_params=pltpu.CompilerParams(
            dimension_semantics=("arbitrary",)),
    )(*args)


def kernel(x, edge_index, prev_hidden, params):
    n, d_in = x.shape
    e = edge_index.shape[1]
    deg = e // n
    idx_t = edge_index[0].reshape(n, deg).T.reshape(-1)

    m1 = _sc_gather(x, idx_t, 200).reshape(deg, n, d_in)
    h1 = _sage_layer(x, m1, params["conv1"], gru=None, relu=True, block=1000)

    m2 = _sc_gather(h1, idx_t, 200).reshape(deg, n, -1)
    h2 = _sage_layer(h1, m2, params["conv2"], gru=None, relu=True, block=1000)

    m3 = _sc_gather(h2, idx_t, 200).reshape(deg, n, -1)
    h3 = _sage_layer(h2, m3, params["conv3"],
                     gru=(prev_hidden, params["gru"]), relu=True, block=1000)

    m4 = _sc_gather(h3, idx_t, 200).reshape(deg, n, -1)
    logits = _sage_layer(h3, m4, params["conv4"], gru=None, relu=False,
                         block=1000)
    d_out = params["conv4"]["W_self"].shape[1]
    return logits[:, :d_out]
